# Initial kernel scaffold; baseline (speedup 1.0000x reference)
import jax
import jax.numpy as jnp
from jax.experimental import pallas as pl


def kernel(x, tables):
    raise NotImplementedError("write your pallas kernel here")


# speedup vs baseline: 1.7528x; 1.7528x over previous
import functools

import numpy as np
import jax
import jax.numpy as jnp
from jax import lax
from jax.experimental import pallas as pl
from jax.experimental.pallas import tpu as pltpu
from jax.experimental.pallas import tpu_sc as plsc

WORLD_MIN = -2.0
WORLD_MAX = 2.0
HASH_SIZE = 2 ** 21
HASH_MASK = HASH_SIZE - 1
NFEAT = 2
NLVL = 20
NCOL = NLVL * NFEAT
SCALE = 10.0
PI2 = np.int32(19349663)
PI3 = np.int32(83492791)
N = 131072
NW = 32
PW = N // NW
CHUNK = 512
NCH = PW // CHUNK
NGRP = CHUNK // 16

_GRIDS = [int(v) for v in np.round(np.geomspace(16, 8192, NLVL))]
_IS_DENSE = [g ** 3 <= HASH_SIZE for g in _GRIDS]
_STAGED = [d and (g ** 3) * NFEAT * 4 <= 90 * 1024
           for g, d in zip(_GRIDS, _IS_DENSE)]


def _floor16(lx):
    it = lx.astype(jnp.int32)
    ft = it.astype(jnp.float32)
    fi = jnp.where(ft > lx, it - 1, it)
    fx = fi.astype(jnp.float32)
    return fi, fx


def _axis(xn, gf):
    lx = xn * gf - 0.5
    fi, fx = _floor16(lx)
    wc = lx - fx
    wf = 1.0 - wc
    return fi, wf, wc


def _hash_idx_w(x16, y16, z16, gf):
    ix0, wfx, wcx = _axis(x16, gf)
    iy0, wfy, wcy = _axis(y16, gf)
    iz0, wfz, wcz = _axis(z16, gf)
    ix1, iy1, iz1 = ix0 + 1, iy0 + 1, iz0 + 1
    hy0, hy1 = iy0 * PI2, iy1 * PI2
    hz0, hz1 = iz0 * PI3, iz1 * PI3
    a = [ix0 ^ hy0, ix0 ^ hy1, ix1 ^ hy0, ix1 ^ hy1]
    wxy = [wfx * wfy, wfx * wcy, wcx * wfy, wcx * wcy]
    wz = [wfz * SCALE, wcz * SCALE]
    idxs, ws = [], []
    for bx in (0, 1):
        for by in (0, 1):
            for bz in (0, 1):
                idxs.append((a[bx * 2 + by] ^ (hz1 if bz else hz0)) & HASH_MASK)
                ws.append(wxy[bx * 2 + by] * wz[bz])
    return idxs, ws


def _dense_idx_w(x16, y16, z16, g):
    gf = float(g)
    out = []
    for v in (x16, y16, z16):
        i0, wf, wc = _axis(v, gf)
        i1 = i0 + 1
        wf = jnp.where(i0 >= 0, wf, 0.0)
        i0 = jnp.maximum(i0, 0)
        wc = jnp.where(i1 <= g - 1, wc, 0.0)
        i1 = jnp.minimum(i1, g - 1)
        out.append((i0, i1, wf, wc))
    (ix0, ix1, wfx, wcx), (iy0, iy1, wfy, wcy), (iz0, iz1, wfz, wcz) = out
    g2 = np.int32(g * g)
    gi = np.int32(g)
    u = [ix0 * g2, ix1 * g2]
    v = [iy0 * gi, iy1 * gi]
    b = [u[0] + v[0], u[0] + v[1], u[1] + v[0], u[1] + v[1]]
    wxy = [wfx * wfy, wfx * wcy, wcx * wfy, wcx * wcy]
    wz = [wfz * SCALE, wcz * SCALE]
    idxs, ws = [], []
    for bx in (0, 1):
        for by in (0, 1):
            for bz in (0, 1):
                idxs.append(b[bx * 2 + by] + (iz1 if bz else iz0))
                ws.append(wxy[bx * 2 + by] * wz[bz])
    return idxs, ws


_MESH = plsc.VectorSubcoreMesh(core_axis_name="c", subcore_axis_name="s")


@functools.partial(
    pl.kernel,
    out_type=jax.ShapeDtypeStruct((N * NCOL,), jnp.float32),
    mesh=_MESH,
    scratch_types=[
        pltpu.VMEM((PW,), jnp.float32),
        pltpu.VMEM((PW,), jnp.float32),
        pltpu.VMEM((PW,), jnp.float32),
        pltpu.VMEM((8 * CHUNK * NFEAT,), jnp.int32),
        pltpu.VMEM((8 * CHUNK,), jnp.float32),
        pltpu.VMEM((8 * CHUNK * NFEAT,), jnp.float32),
        pltpu.VMEM((CHUNK * NCOL,), jnp.float32),
        pltpu.VMEM((_GRIDS[0] ** 3 * NFEAT,), jnp.float32),
        pltpu.VMEM((_GRIDS[1] ** 3 * NFEAT,), jnp.float32),
        pltpu.SemaphoreType.DMA,
    ],
    compiler_params=pltpu.CompilerParams(needs_layout_passes=False),
)
def _sc_encode(*args):
    xs, ys, zs = args[0], args[1], args[2]
    tabs = args[3:3 + NLVL]
    stflat = args[3 + NLVL:5 + NLVL]
    out = args[5 + NLVL]
    xb, yb, zb, idxb, wb, rows, outb, st0, st1, sem = args[6 + NLVL:]
    staged_refs = {0: st0, 1: st1}

    wid = lax.axis_index("c") * 16 + lax.axis_index("s")
    base = wid * PW
    pltpu.sync_copy(xs.at[pl.ds(base, PW)], xb)
    pltpu.sync_copy(ys.at[pl.ds(base, PW)], yb)
    pltpu.sync_copy(zs.at[pl.ds(base, PW)], zb)
    pltpu.sync_copy(stflat[0], st0)
    pltpu.sync_copy(stflat[1], st1)

    inv = 1.0 / (WORLD_MAX - WORLD_MIN)

    def _norm(j, carry):
        for b in (xb, yb, zb):
            v = b[pl.ds(j * 16, 16)]
            b[pl.ds(j * 16, 16)] = (v - WORLD_MIN) * inv
        return carry

    lax.fori_loop(0, PW // 16, _norm, 0)

    iota = lax.iota(jnp.int32, 16)
    CF = CHUNK * NFEAT

    def _chunk(ch, carry):
        cb = ch * CHUNK
        for lvl in range(NLVL):
            g = _GRIDS[lvl]
            dense = _IS_DENSE[lvl]

            if _STAGED[lvl]:
                st = staged_refs[lvl]

                def _grp_fused(i, c, lvl=lvl, g=g, st=st, cb=cb):
                    off = cb + i * 16
                    x16 = xb[pl.ds(off, 16)]
                    y16 = yb[pl.ds(off, 16)]
                    z16 = zb[pl.ds(off, 16)]
                    idxs, ws = _dense_idx_w(x16, y16, z16, g)
                    ob = (i * 16 + iota) * NCOL + (2 * lvl)
                    for m in range(8):
                        e2 = idxs[m] * NFEAT
                        for f in range(NFEAT):
                            val = plsc.load_gather(st, [e2 + f])
                            contrib = val * ws[m]
                            if m == 0:
                                plsc.store_scatter(outb, [ob + f], contrib)
                            else:
                                plsc.addupdate_scatter(outb, [ob + f], contrib)
                    return c

                lax.fori_loop(0, NGRP, _grp_fused, 0)
            else:
                def _grp_a(i, c, lvl=lvl, g=g, dense=dense, cb=cb):
                    off = cb + i * 16
                    x16 = xb[pl.ds(off, 16)]
                    y16 = yb[pl.ds(off, 16)]
                    z16 = zb[pl.ds(off, 16)]
                    if dense:
                        idxs, ws = _dense_idx_w(x16, y16, z16, g)
                    else:
                        idxs, ws = _hash_idx_w(x16, y16, z16, float(g))
                    for m in range(8):
                        e2 = idxs[m] * NFEAT
                        for f in range(NFEAT):
                            idxb[pl.ds(m * CF + f * CHUNK + i * 16, 16)] = (
                                e2 + f)
                        wb[pl.ds(m * CHUNK + i * 16, 16)] = ws[m]
                    return c

                lax.fori_loop(0, NGRP, _grp_a, 0)

                copies = [
                    pltpu.async_copy(
                        tabs[lvl].at[idxb.at[pl.ds(m * CF, CF)]],
                        rows.at[pl.ds(m * CF, CF)],
                        sem)
                    for m in range(8)
                ]
                for cpy in copies:
                    cpy.wait()

                def _grp_acc(i, c, lvl=lvl):
                    pid = i * 16 + iota
                    ob = pid * NCOL + (2 * lvl)
                    for m in range(8):
                        w16 = wb[pl.ds(m * CHUNK + i * 16, 16)]
                        for f in range(NFEAT):
                            val = rows[pl.ds(m * CF + f * CHUNK + i * 16, 16)]
                            contrib = val * w16
                            if m == 0:
                                plsc.store_scatter(outb, [ob + f], contrib)
                            else:
                                plsc.addupdate_scatter(outb, [ob + f], contrib)
                    return c

                lax.fori_loop(0, NGRP, _grp_acc, 0)

        pltpu.sync_copy(outb, out.at[pl.ds((base + cb) * NCOL, CHUNK * NCOL)])
        return carry

    lax.fori_loop(0, NCH, _chunk, 0)


def kernel(x, tables):
    xs = x[:, 0]
    ys = x[:, 1]
    zs = x[:, 2]
    flat = [t.reshape(-1) for t in tables]
    staged = [t.reshape(-1) for t, s in zip(tables, _STAGED) if s]
    return _sc_encode(xs, ys, zs, *flat, *staged).reshape(N, NCOL)


# speedup vs baseline: 14.3861x; 8.2074x over previous
import functools

import numpy as np
import jax
import jax.numpy as jnp
from jax import lax
from jax.experimental import pallas as pl
from jax.experimental.pallas import tpu as pltpu
from jax.experimental.pallas import tpu_sc as plsc

WORLD_MIN = -2.0
WORLD_MAX = 2.0
HASH_SIZE = 2 ** 21
HASH_MASK = HASH_SIZE - 1
NFEAT = 2
NLVL = 20
NCOL = NLVL * NFEAT
SCALE = 10.0
PI2 = np.int32(19349663)
PI3 = np.int32(83492791)
N = 131072
NW = 32
PW = N // NW
CHUNK = 512
NCH = PW // CHUNK
NGRP = CHUNK // 16
CF = CHUNK * NFEAT

_GRIDS = [int(v) for v in np.round(np.geomspace(16, 8192, NLVL))]
_IS_DENSE = [g ** 3 <= HASH_SIZE for g in _GRIDS]
_STAGED = [d and (g ** 3) * NFEAT * 4 <= 90 * 1024
           for g, d in zip(_GRIDS, _IS_DENSE)]


def _floor16(lx):
    it = lx.astype(jnp.int32)
    ft = it.astype(jnp.float32)
    fi = jnp.where(ft > lx, it - 1, it)
    fx = fi.astype(jnp.float32)
    return fi, fx


def _axis(xn, gf):
    lx = xn * gf - 0.5
    fi, fx = _floor16(lx)
    wc = lx - fx
    wf = 1.0 - wc
    return fi, wf, wc


def _hash_idx_w(x16, y16, z16, gf):
    ix0, wfx, wcx = _axis(x16, gf)
    iy0, wfy, wcy = _axis(y16, gf)
    iz0, wfz, wcz = _axis(z16, gf)
    ix1, iy1, iz1 = ix0 + 1, iy0 + 1, iz0 + 1
    hy0, hy1 = iy0 * PI2, iy1 * PI2
    hz0, hz1 = iz0 * PI3, iz1 * PI3
    a = [ix0 ^ hy0, ix0 ^ hy1, ix1 ^ hy0, ix1 ^ hy1]
    wxy = [wfx * wfy, wfx * wcy, wcx * wfy, wcx * wcy]
    wz = [wfz * SCALE, wcz * SCALE]
    idxs, ws = [], []
    for bx in (0, 1):
        for by in (0, 1):
            for bz in (0, 1):
                h = (a[bx * 2 + by] ^ (hz1 if bz else hz0)) & HASH_MASK
                idxs.append(h + ((h >> 7) << 7))
                ws.append(wxy[bx * 2 + by] * wz[bz])
    return idxs, ws


def _dense_idx_w(x16, y16, z16, g):
    gf = float(g)
    out = []
    for v in (x16, y16, z16):
        i0, wf, wc = _axis(v, gf)
        i1 = i0 + 1
        wf = jnp.where(i0 >= 0, wf, 0.0)
        i0 = jnp.maximum(i0, 0)
        wc = jnp.where(i1 <= g - 1, wc, 0.0)
        i1 = jnp.minimum(i1, g - 1)
        out.append((i0, i1, wf, wc))
    (ix0, ix1, wfx, wcx), (iy0, iy1, wfy, wcy), (iz0, iz1, wfz, wcz) = out
    g2 = np.int32(g * g * NFEAT)
    gi = np.int32(g * NFEAT)
    u = [ix0 * g2, ix1 * g2]
    v = [iy0 * gi, iy1 * gi]
    b = [u[0] + v[0], u[0] + v[1], u[1] + v[0], u[1] + v[1]]
    z = [iz0 * NFEAT, iz1 * NFEAT]
    wxy = [wfx * wfy, wfx * wcy, wcx * wfy, wcx * wcy]
    wz = [wfz * SCALE, wcz * SCALE]
    idxs, ws = [], []
    for bx in (0, 1):
        for by in (0, 1):
            for bz in (0, 1):
                idxs.append(b[bx * 2 + by] + z[bz])
                ws.append(wxy[bx * 2 + by] * wz[bz])
    return idxs, ws


_MESH = plsc.VectorSubcoreMesh(core_axis_name="c", subcore_axis_name="s")


@functools.partial(
    pl.kernel,
    out_type=jax.ShapeDtypeStruct((N * NCOL,), jnp.float32),
    mesh=_MESH,
    scratch_types=[
        pltpu.VMEM((PW,), jnp.float32),
        pltpu.VMEM((PW,), jnp.float32),
        pltpu.VMEM((PW,), jnp.float32),
        pltpu.VMEM((8 * CF,), jnp.int32),
        pltpu.VMEM((8 * CHUNK,), jnp.float32),
        pltpu.VMEM((8 * CF,), jnp.float32),
        pltpu.VMEM((CHUNK * NCOL,), jnp.float32),
        pltpu.VMEM((_GRIDS[0] ** 3 * NFEAT,), jnp.float32),
        pltpu.VMEM((_GRIDS[1] ** 3 * NFEAT,), jnp.float32),
        pltpu.SemaphoreType.DMA,
    ],
    compiler_params=pltpu.CompilerParams(needs_layout_passes=False),
)
def _sc_encode(*args):
    xs, ys, zs = args[0], args[1], args[2]
    tabs = args[3:3 + NLVL]
    out = args[3 + NLVL]
    xb, yb, zb, idxb, wb, rows, outb, st0, st1, sem = args[4 + NLVL:]
    staged_refs = {0: st0, 1: st1}

    wid = lax.axis_index("c") * 16 + lax.axis_index("s")
    base = wid * PW
    pltpu.sync_copy(xs.at[pl.ds(base, PW)], xb)
    pltpu.sync_copy(ys.at[pl.ds(base, PW)], yb)
    pltpu.sync_copy(zs.at[pl.ds(base, PW)], zb)
    pltpu.sync_copy(tabs[0], st0)
    pltpu.sync_copy(tabs[1], st1)

    inv = 1.0 / (WORLD_MAX - WORLD_MIN)

    def _norm(j, carry):
        for b in (xb, yb, zb):
            v = b[pl.ds(j * 16, 16)]
            b[pl.ds(j * 16, 16)] = (v - WORLD_MIN) * inv
        return carry

    lax.fori_loop(0, PW // 16, _norm, 0)

    def _chunk(ch, carry):
        cb = ch * CHUNK
        for lvl in range(NLVL):
            g = _GRIDS[lvl]
            dense = _IS_DENSE[lvl]
            fstride = 1 if dense else 128
            coff = [((2 * lvl + f) >> 3) * (8 * CHUNK)
                    + ((2 * lvl + f) & 7) * 128 for f in range(NFEAT)]

            if _STAGED[lvl]:
                st = staged_refs[lvl]

                def _grp_fused(i, c, g=g, st=st, cb=cb, coff=coff):
                    off = cb + i * 16
                    x16 = xb[pl.ds(off, 16)]
                    y16 = yb[pl.ds(off, 16)]
                    z16 = zb[pl.ds(off, 16)]
                    idxs, ws = _dense_idx_w(x16, y16, z16, g)
                    ob = (i >> 3) * 1024 + (i & 7) * 16
                    for f in range(NFEAT):
                        acc = None
                        for m in range(8):
                            val = plsc.load_gather(st, [idxs[m] + f])
                            acc = (val * ws[m] if acc is None
                                   else acc + val * ws[m])
                        outb[pl.ds(ob + coff[f], 16)] = acc
                    return c

                lax.fori_loop(0, NGRP, _grp_fused, 0)
            else:
                def _grp_a(i, c, g=g, dense=dense, cb=cb, fstride=fstride):
                    off = cb + i * 16
                    x16 = xb[pl.ds(off, 16)]
                    y16 = yb[pl.ds(off, 16)]
                    z16 = zb[pl.ds(off, 16)]
                    if dense:
                        idxs, ws = _dense_idx_w(x16, y16, z16, g)
                    else:
                        idxs, ws = _hash_idx_w(x16, y16, z16, float(g))
                    for m in range(8):
                        idxb[pl.ds(m * CF + i * 16, 16)] = idxs[m]
                        idxb[pl.ds(m * CF + CHUNK + i * 16, 16)] = (
                            idxs[m] + fstride)
                        wb[pl.ds(m * CHUNK + i * 16, 16)] = ws[m]
                    return c

                lax.fori_loop(0, NGRP, _grp_a, 0)

                copies = [
                    pltpu.async_copy(
                        tabs[lvl].at[idxb.at[pl.ds(m * CF, CF)]],
                        rows.at[pl.ds(m * CF, CF)],
                        sem)
                    for m in range(8)
                ]
                for cpy in copies:
                    cpy.wait()

                def _grp_acc(i, c, coff=coff):
                    ob = (i >> 3) * 1024 + (i & 7) * 16
                    w16 = [wb[pl.ds(m * CHUNK + i * 16, 16)]
                           for m in range(8)]
                    for f in range(NFEAT):
                        acc = None
                        for m in range(8):
                            val = rows[pl.ds(m * CF + f * CHUNK + i * 16, 16)]
                            acc = (val * w16[m] if acc is None
                                   else acc + val * w16[m])
                        outb[pl.ds(ob + coff[f], 16)] = acc
                    return c

                lax.fori_loop(0, NGRP, _grp_acc, 0)

        p0 = base + cb
        for cblk in range(NCOL // 8):
            pltpu.sync_copy(
                outb.at[pl.ds(cblk * (8 * CHUNK), 8 * CHUNK)],
                out.at[pl.ds(cblk * (N * 8) + p0 * 8, 8 * CHUNK)])
        return carry

    lax.fori_loop(0, NCH, _chunk, 0)


def kernel(x, tables):
    xs = x[:, 0]
    ys = x[:, 1]
    zs = x[:, 2]
    flat = []
    for t, g, d in zip(tables, _GRIDS, _IS_DENSE):
        if d:
            flat.append(t.reshape(-1))
        else:
            flat.append(t.reshape(HASH_SIZE // 128, 128, NFEAT)
                         .transpose(0, 2, 1).reshape(-1))
    out = _sc_encode(xs, ys, zs, *flat)
    return (out.reshape(NCOL // 8, N // 128, 8, 128)
               .transpose(1, 3, 0, 2).reshape(N, NCOL))


# speedup vs baseline: 29.3802x; 2.0423x over previous
import functools

import numpy as np
import jax
import jax.numpy as jnp
from jax import lax
from jax.experimental import pallas as pl
from jax.experimental.pallas import tpu as pltpu
from jax.experimental.pallas import tpu_sc as plsc

WORLD_MIN = -2.0
WORLD_MAX = 2.0
HASH_SIZE = 2 ** 21
HASH_MASK = HASH_SIZE - 1
NFEAT = 2
NLVL = 20
NCOL = NLVL * NFEAT
SCALE = 10.0
PI2 = np.int32(19349663)
PI3 = np.int32(83492791)
N = 131072
NW = 32
PW = N // NW
CHUNK = 512
NCH = PW // CHUNK
NGRP = CHUNK // 16
CF = CHUNK * NFEAT

_GRIDS = [int(v) for v in np.round(np.geomspace(16, 8192, NLVL))]
_IS_DENSE = [g ** 3 <= HASH_SIZE for g in _GRIDS]
_STAGED = [d and (g * g * 256 * 4) <= 300 * 1024
           for g, d in zip(_GRIDS, _IS_DENSE)]


def _floor16(lx):
    it = lx.astype(jnp.int32)
    ft = it.astype(jnp.float32)
    fi = jnp.where(ft > lx, it - 1, it)
    fx = fi.astype(jnp.float32)
    return fi, fx


def _axis(xn, gf):
    lx = xn * gf - 0.5
    fi, fx = _floor16(lx)
    wc = lx - fx
    wf = 1.0 - wc
    return fi, wf, wc


def _hash_idx_w(x16, y16, z16, gf):
    ix0, wfx, wcx = _axis(x16, gf)
    iy0, wfy, wcy = _axis(y16, gf)
    iz0, wfz, wcz = _axis(z16, gf)
    ix1, iy1, iz1 = ix0 + 1, iy0 + 1, iz0 + 1
    hy0, hy1 = iy0 * PI2, iy1 * PI2
    hz0, hz1 = iz0 * PI3, iz1 * PI3
    a = [ix0 ^ hy0, ix0 ^ hy1, ix1 ^ hy0, ix1 ^ hy1]
    wxy = [wfx * wfy, wfx * wcy, wcx * wfy, wcx * wcy]
    wz = [wfz * SCALE, wcz * SCALE]
    idxs, ws = [], []
    for bx in (0, 1):
        for by in (0, 1):
            for bz in (0, 1):
                h = (a[bx * 2 + by] ^ (hz1 if bz else hz0)) & HASH_MASK
                idxs.append(h + ((h >> 7) << 7))
                ws.append(wxy[bx * 2 + by] * wz[bz])
    return idxs, ws


def _dense_idx_w(x16, y16, z16, g):
    gf = float(g)
    out = []
    for v in (x16, y16, z16):
        i0, wf, wc = _axis(v, gf)
        i1 = i0 + 1
        wf = jnp.where(i0 >= 0, wf, 0.0)
        i0 = jnp.maximum(i0, 0)
        wc = jnp.where(i1 <= g - 1, wc, 0.0)
        i1 = jnp.minimum(i1, g - 1)
        out.append((i0, i1, wf, wc))
    (ix0, ix1, wfx, wcx), (iy0, iy1, wfy, wcy), (iz0, iz1, wfz, wcz) = out
    g2 = np.int32(g * 256)
    gi = np.int32(256)
    u = [ix0 * g2, ix1 * g2]
    v = [iy0 * gi, iy1 * gi]
    b = [u[0] + v[0], u[0] + v[1], u[1] + v[0], u[1] + v[1]]
    wxy = [wfx * wfy, wfx * wcy, wcx * wfy, wcx * wcy]
    wz = [wfz * SCALE, wcz * SCALE]
    idxs, ws = [], []
    for bx in (0, 1):
        for by in (0, 1):
            for bz in (0, 1):
                idxs.append(b[bx * 2 + by] + (iz1 if bz else iz0))
                ws.append(wxy[bx * 2 + by] * wz[bz])
    return idxs, ws


_MESH = plsc.VectorSubcoreMesh(core_axis_name="c", subcore_axis_name="s")


@functools.partial(
    pl.kernel,
    out_type=jax.ShapeDtypeStruct((N * NCOL,), jnp.float32),
    mesh=_MESH,
    scratch_types=[
        pltpu.VMEM((PW,), jnp.float32),
        pltpu.VMEM((PW,), jnp.float32),
        pltpu.VMEM((PW,), jnp.float32),
        pltpu.VMEM((8 * CF,), jnp.int32),
        pltpu.VMEM((8 * CHUNK,), jnp.float32),
        pltpu.VMEM((8 * CF,), jnp.float32),
        pltpu.VMEM((CHUNK * NCOL,), jnp.float32),
        pltpu.VMEM((_GRIDS[0] ** 2 * 256,), jnp.float32),
        pltpu.SemaphoreType.DMA,
    ],
    compiler_params=pltpu.CompilerParams(needs_layout_passes=False),
)
def _sc_encode(*args):
    xs, ys, zs = args[0], args[1], args[2]
    tabs = args[3:3 + NLVL]
    out = args[3 + NLVL]
    xb, yb, zb, idxb, wb, rows, outb, st0, sem = args[4 + NLVL:]
    staged_refs = {0: st0}

    wid = lax.axis_index("c") * 16 + lax.axis_index("s")
    base = wid * PW
    pltpu.sync_copy(xs.at[pl.ds(base, PW)], xb)
    pltpu.sync_copy(ys.at[pl.ds(base, PW)], yb)
    pltpu.sync_copy(zs.at[pl.ds(base, PW)], zb)
    pltpu.sync_copy(tabs[0], st0)

    inv = 1.0 / (WORLD_MAX - WORLD_MIN)

    def _norm(j, carry):
        for b in (xb, yb, zb):
            v = b[pl.ds(j * 16, 16)]
            b[pl.ds(j * 16, 16)] = (v - WORLD_MIN) * inv
        return carry

    lax.fori_loop(0, PW // 16, _norm, 0)

    def _chunk(ch, carry):
        cb = ch * CHUNK
        for lvl in range(NLVL):
            g = _GRIDS[lvl]
            dense = _IS_DENSE[lvl]
            coff = [((2 * lvl + f) >> 3) * (8 * CHUNK)
                    + ((2 * lvl + f) & 7) * 128 for f in range(NFEAT)]

            if _STAGED[lvl]:
                st = staged_refs[lvl]

                def _grp_fused(i, c, g=g, st=st, cb=cb, coff=coff):
                    off = cb + i * 16
                    x16 = xb[pl.ds(off, 16)]
                    y16 = yb[pl.ds(off, 16)]
                    z16 = zb[pl.ds(off, 16)]
                    idxs, ws = _dense_idx_w(x16, y16, z16, g)
                    ob = (i >> 3) * 1024 + (i & 7) * 16
                    for f in range(NFEAT):
                        acc = None
                        for m in range(8):
                            val = plsc.load_gather(st, [idxs[m] + f * 128])
                            acc = (val * ws[m] if acc is None
                                   else acc + val * ws[m])
                        outb[pl.ds(ob + coff[f], 16)] = acc
                    return c

                lax.fori_loop(0, NGRP, _grp_fused, 0)
            else:
                def _grp_a(i, c, g=g, dense=dense, cb=cb):
                    off = cb + i * 16
                    x16 = xb[pl.ds(off, 16)]
                    y16 = yb[pl.ds(off, 16)]
                    z16 = zb[pl.ds(off, 16)]
                    if dense:
                        idxs, ws = _dense_idx_w(x16, y16, z16, g)
                    else:
                        idxs, ws = _hash_idx_w(x16, y16, z16, float(g))
                    for m in range(8):
                        idxb[pl.ds(m * CF + i * 16, 16)] = idxs[m]
                        idxb[pl.ds(m * CF + CHUNK + i * 16, 16)] = (
                            idxs[m] + 128)
                        wb[pl.ds(m * CHUNK + i * 16, 16)] = ws[m]
                    return c

                lax.fori_loop(0, NGRP, _grp_a, 0)

                copies = [
                    pltpu.async_copy(
                        tabs[lvl].at[idxb.at[pl.ds(m * CF, CF)]],
                        rows.at[pl.ds(m * CF, CF)],
                        sem)
                    for m in range(8)
                ]
                for cpy in copies:
                    cpy.wait()

                def _grp_acc(i, c, coff=coff):
                    ob = (i >> 3) * 1024 + (i & 7) * 16
                    w16 = [wb[pl.ds(m * CHUNK + i * 16, 16)]
                           for m in range(8)]
                    for f in range(NFEAT):
                        acc = None
                        for m in range(8):
                            val = rows[pl.ds(m * CF + f * CHUNK + i * 16, 16)]
                            acc = (val * w16[m] if acc is None
                                   else acc + val * w16[m])
                        outb[pl.ds(ob + coff[f], 16)] = acc
                    return c

                lax.fori_loop(0, NGRP, _grp_acc, 0)

        p0 = base + cb
        for cblk in range(NCOL // 8):
            pltpu.sync_copy(
                outb.at[pl.ds(cblk * (8 * CHUNK), 8 * CHUNK)],
                out.at[pl.ds(cblk * (N * 8) + p0 * 8, 8 * CHUNK)])
        return carry

    lax.fori_loop(0, NCH, _chunk, 0)


def kernel(x, tables):
    xs = x[:, 0]
    ys = x[:, 1]
    zs = x[:, 2]
    flat = []
    for t, g, d in zip(tables, _GRIDS, _IS_DENSE):
        if d:
            flat.append(jnp.pad(t.transpose(0, 1, 3, 2),
                                ((0, 0), (0, 0), (0, 0), (0, 128 - g)))
                        .reshape(-1))
        else:
            flat.append(t.reshape(HASH_SIZE // 128, 128, NFEAT)
                         .transpose(0, 2, 1).reshape(-1))
    out = _sc_encode(xs, ys, zs, *flat)
    return (out.reshape(NCOL // 8, N // 128, 8, 128)
               .transpose(1, 3, 0, 2).reshape(N, NCOL))


# speedup vs baseline: 32.7206x; 1.1137x over previous
import functools

import numpy as np
import jax
import jax.numpy as jnp
from jax import lax
from jax.experimental import pallas as pl
from jax.experimental.pallas import tpu as pltpu
from jax.experimental.pallas import tpu_sc as plsc

WORLD_MIN = -2.0
WORLD_MAX = 2.0
HASH_SIZE = 2 ** 21
HASH_MASK = HASH_SIZE - 1
NFEAT = 2
NLVL = 20
NCOL = NLVL * NFEAT
SCALE = 10.0
PI2 = np.int32(19349663)
PI3 = np.int32(83492791)
N = 131072
NW = 32
PW = N // NW
CHUNK = 512
NCH = PW // CHUNK
NGRP = CHUNK // 16
CF = CHUNK * NFEAT

_GRIDS = [int(v) for v in np.round(np.geomspace(16, 8192, NLVL))]
_IS_DENSE = [g ** 3 <= HASH_SIZE for g in _GRIDS]
_STAGED = [False] * NLVL


def _floor16(lx):
    it = lx.astype(jnp.int32)
    ft = it.astype(jnp.float32)
    fi = jnp.where(ft > lx, it - 1, it)
    fx = fi.astype(jnp.float32)
    return fi, fx


def _axis(xn, gf):
    lx = xn * gf - 0.5
    fi, fx = _floor16(lx)
    wc = lx - fx
    wf = 1.0 - wc
    return fi, wf, wc


def _hash_idx_w(x16, y16, z16, gf):
    ix0, wfx, wcx = _axis(x16, gf)
    iy0, wfy, wcy = _axis(y16, gf)
    iz0, wfz, wcz = _axis(z16, gf)
    ix1, iy1, iz1 = ix0 + 1, iy0 + 1, iz0 + 1
    hy0, hy1 = iy0 * PI2, iy1 * PI2
    hz0, hz1 = iz0 * PI3, iz1 * PI3
    a = [ix0 ^ hy0, ix0 ^ hy1, ix1 ^ hy0, ix1 ^ hy1]
    wxy = [wfx * wfy, wfx * wcy, wcx * wfy, wcx * wcy]
    wz = [wfz * SCALE, wcz * SCALE]
    idxs, ws = [], []
    for bx in (0, 1):
        for by in (0, 1):
            for bz in (0, 1):
                h = (a[bx * 2 + by] ^ (hz1 if bz else hz0)) & HASH_MASK
                idxs.append(h + (h & np.int32(-128)))
                ws.append(wxy[bx * 2 + by] * wz[bz])
    return idxs, ws


def _dense_idx_w(x16, y16, z16, g):
    gf = float(g)
    out = []
    for v in (x16, y16, z16):
        i0, wf, wc = _axis(v, gf)
        i1 = i0 + 1
        wf = jnp.where(i0 >= 0, wf, 0.0)
        i0 = jnp.maximum(i0, 0)
        wc = jnp.where(i1 <= g - 1, wc, 0.0)
        i1 = jnp.minimum(i1, g - 1)
        out.append((i0, i1, wf, wc))
    (ix0, ix1, wfx, wcx), (iy0, iy1, wfy, wcy), (iz0, iz1, wfz, wcz) = out
    g2 = np.int32(g * 256)
    gi = np.int32(256)
    u = [ix0 * g2, ix1 * g2]
    v = [iy0 * gi, iy1 * gi]
    b = [u[0] + v[0], u[0] + v[1], u[1] + v[0], u[1] + v[1]]
    wxy = [wfx * wfy, wfx * wcy, wcx * wfy, wcx * wcy]
    wz = [wfz * SCALE, wcz * SCALE]
    idxs, ws = [], []
    for bx in (0, 1):
        for by in (0, 1):
            for bz in (0, 1):
                idxs.append(b[bx * 2 + by] + (iz1 if bz else iz0))
                ws.append(wxy[bx * 2 + by] * wz[bz])
    return idxs, ws


_MESH = plsc.VectorSubcoreMesh(core_axis_name="c", subcore_axis_name="s")


@functools.partial(
    pl.kernel,
    out_type=jax.ShapeDtypeStruct((N * NCOL,), jnp.float32),
    mesh=_MESH,
    scratch_types=[
        pltpu.VMEM((PW,), jnp.float32),
        pltpu.VMEM((PW,), jnp.float32),
        pltpu.VMEM((PW,), jnp.float32),
        pltpu.VMEM((8 * CF,), jnp.int32),
        pltpu.VMEM((8 * CF,), jnp.int32),
        pltpu.VMEM((8 * CHUNK,), jnp.float32),
        pltpu.VMEM((8 * CHUNK,), jnp.float32),
        pltpu.VMEM((8 * CF,), jnp.float32),
        pltpu.VMEM((8 * CF,), jnp.float32),
        pltpu.VMEM((CHUNK * NCOL,), jnp.float32),
        pltpu.SemaphoreType.DMA,
    ],
    compiler_params=pltpu.CompilerParams(needs_layout_passes=False),
)
def _sc_encode(*args):
    xs, ys, zs = args[0], args[1], args[2]
    tabs = args[3:3 + NLVL]
    out = args[3 + NLVL]
    (xb, yb, zb, idxb0, idxb1, wb0, wb1, rows0, rows1, outb,
     sem) = args[4 + NLVL:]
    idxbs, wbs, rowss = [idxb0, idxb1], [wb0, wb1], [rows0, rows1]

    wid = lax.axis_index("c") * 16 + lax.axis_index("s")
    base = wid * PW
    pltpu.sync_copy(xs.at[pl.ds(base, PW)], xb)
    pltpu.sync_copy(ys.at[pl.ds(base, PW)], yb)
    pltpu.sync_copy(zs.at[pl.ds(base, PW)], zb)

    inv = 1.0 / (WORLD_MAX - WORLD_MIN)

    def _norm(j, carry):
        for b in (xb, yb, zb):
            v = b[pl.ds(j * 16, 16)]
            b[pl.ds(j * 16, 16)] = (v - WORLD_MIN) * inv
        return carry

    lax.fori_loop(0, PW // 16, _norm, 0)

    def _coff(lvl):
        return [((2 * lvl + f) >> 3) * (8 * CHUNK)
                + ((2 * lvl + f) & 7) * 128 for f in range(NFEAT)]

    _DMA_LVLS = [lvl for lvl in range(NLVL) if not _STAGED[lvl]]
    ND = len(_DMA_LVLS)

    def _chunk(ch, carry):
        cb = ch * CHUNK

        def _phase_a(k):
            lvl = _DMA_LVLS[k]
            g, dense = _GRIDS[lvl], _IS_DENSE[lvl]
            idxb, wb = idxbs[k % 2], wbs[k % 2]

            def _grp_a(i, c, g=g, dense=dense, cb=cb, idxb=idxb, wb=wb):
                off = cb + i * 16
                x16 = xb[pl.ds(off, 16)]
                y16 = yb[pl.ds(off, 16)]
                z16 = zb[pl.ds(off, 16)]
                if dense:
                    idxs, ws = _dense_idx_w(x16, y16, z16, g)
                else:
                    idxs, ws = _hash_idx_w(x16, y16, z16, float(g))
                for m in range(8):
                    idxb[pl.ds(m * CF + i * 16, 16)] = idxs[m]
                    idxb[pl.ds(m * CF + CHUNK + i * 16, 16)] = idxs[m] + 128
                    wb[pl.ds(m * CHUNK + i * 16, 16)] = ws[m]
                return c

            lax.fori_loop(0, NGRP, _grp_a, 0)

        def _fire(k):
            lvl = _DMA_LVLS[k]
            idxb, rows = idxbs[k % 2], rowss[k % 2]
            return [
                pltpu.async_copy(
                    tabs[lvl].at[idxb.at[pl.ds(m * CF, CF)]],
                    rows.at[pl.ds(m * CF, CF)],
                    sem)
                for m in range(8)
            ]

        def _acc(k):
            lvl = _DMA_LVLS[k]
            coff = _coff(lvl)
            wb, rows = wbs[k % 2], rowss[k % 2]

            def _grp_acc(i, c, coff=coff, wb=wb, rows=rows):
                ob = (i >> 3) * 1024 + (i & 7) * 16
                w16 = [wb[pl.ds(m * CHUNK + i * 16, 16)] for m in range(8)]
                for f in range(NFEAT):
                    acc = None
                    for m in range(8):
                        val = rows[pl.ds(m * CF + f * CHUNK + i * 16, 16)]
                        acc = (val * w16[m] if acc is None
                               else acc + val * w16[m])
                    outb[pl.ds(ob + coff[f], 16)] = acc
                return c

            lax.fori_loop(0, NGRP, _grp_acc, 0)

        _phase_a(0)
        pending = {0: _fire(0)}
        _phase_a(1)
        for k in range(ND):
            for cpy in pending.pop(k):
                cpy.wait()
            if k + 1 < ND:
                pending[k + 1] = _fire(k + 1)
            _acc(k)
            if k + 2 < ND:
                _phase_a(k + 2)

        p0 = base + cb
        for cblk in range(NCOL // 8):
            pltpu.sync_copy(
                outb.at[pl.ds(cblk * (8 * CHUNK), 8 * CHUNK)],
                out.at[pl.ds(cblk * (N * 8) + p0 * 8, 8 * CHUNK)])
        return carry

    lax.fori_loop(0, NCH, _chunk, 0)


def kernel(x, tables):
    xs = x[:, 0]
    ys = x[:, 1]
    zs = x[:, 2]
    flat = []
    for t, g, d in zip(tables, _GRIDS, _IS_DENSE):
        if d:
            flat.append(jnp.pad(t.transpose(0, 1, 3, 2),
                                ((0, 0), (0, 0), (0, 0), (0, 128 - g)))
                        .reshape(-1))
        else:
            flat.append(t.reshape(HASH_SIZE // 128, 128, NFEAT)
                         .transpose(0, 2, 1).reshape(-1))
    out = _sc_encode(xs, ys, zs, *flat)
    return (out.reshape(NCOL // 8, N // 128, 8, 128)
               .transpose(1, 3, 0, 2).reshape(N, NCOL))


# speedup vs baseline: 35.1469x; 1.0742x over previous
import functools

import numpy as np
import jax
import jax.numpy as jnp
from jax import lax
from jax.experimental import pallas as pl
from jax.experimental.pallas import tpu as pltpu
from jax.experimental.pallas import tpu_sc as plsc

WORLD_MIN = -2.0
WORLD_MAX = 2.0
HASH_SIZE = 2 ** 21
HASH_MASK = HASH_SIZE - 1
NFEAT = 2
NLVL = 20
NCOL = NLVL * NFEAT
SCALE = 10.0
PI2 = np.int32(19349663)
PI3 = np.int32(83492791)
N = 131072
NW = 32
PW = N // NW
CHUNK = 512
NCH = PW // CHUNK
NGRP = CHUNK // 16
CF = CHUNK * NFEAT

_GRIDS = [int(v) for v in np.round(np.geomspace(16, 8192, NLVL))]
_IS_DENSE = [g ** 3 <= HASH_SIZE for g in _GRIDS]
_STAGED = [False] * NLVL


def _floor16(lx):
    it = lx.astype(jnp.int32)
    ft = it.astype(jnp.float32)
    fi = jnp.where(ft > lx, it - 1, it)
    fx = fi.astype(jnp.float32)
    return fi, fx


def _axis(xn, gf):
    lx = xn * gf - 0.5
    fi, fx = _floor16(lx)
    wc = lx - fx
    wf = 1.0 - wc
    return fi, wf, wc


def _hash_idx_w(x16, y16, z16, gf):
    ix0, wfx, wcx = _axis(x16, gf)
    iy0, wfy, wcy = _axis(y16, gf)
    iz0, wfz, wcz = _axis(z16, gf)
    ix1, iy1, iz1 = ix0 + 1, iy0 + 1, iz0 + 1
    hy0, hy1 = iy0 * PI2, iy1 * PI2
    hz0, hz1 = iz0 * PI3, iz1 * PI3
    a = [ix0 ^ hy0, ix0 ^ hy1, ix1 ^ hy0, ix1 ^ hy1]
    wxy = [wfx * wfy, wfx * wcy, wcx * wfy, wcx * wcy]
    wz = [wfz * SCALE, wcz * SCALE]
    idxs, ws = [], []
    for bx in (0, 1):
        for by in (0, 1):
            for bz in (0, 1):
                h = (a[bx * 2 + by] ^ (hz1 if bz else hz0)) & HASH_MASK
                idxs.append(h + (h & np.int32(-128)))
                ws.append(wxy[bx * 2 + by] * wz[bz])
    return idxs, ws


def _dense_idx_w(x16, y16, z16, g):
    gf = float(g)
    out = []
    for v in (x16, y16, z16):
        i0, wf, wc = _axis(v, gf)
        i1 = i0 + 1
        wf = jnp.where(i0 >= 0, wf, 0.0)
        i0 = jnp.maximum(i0, 0)
        wc = jnp.where(i1 <= g - 1, wc, 0.0)
        i1 = jnp.minimum(i1, g - 1)
        out.append((i0, i1, wf, wc))
    (ix0, ix1, wfx, wcx), (iy0, iy1, wfy, wcy), (iz0, iz1, wfz, wcz) = out
    g2 = np.int32(g * 256)
    gi = np.int32(256)
    u = [ix0 * g2, ix1 * g2]
    v = [iy0 * gi, iy1 * gi]
    b = [u[0] + v[0], u[0] + v[1], u[1] + v[0], u[1] + v[1]]
    wxy = [wfx * wfy, wfx * wcy, wcx * wfy, wcx * wcy]
    wz = [wfz * SCALE, wcz * SCALE]
    idxs, ws = [], []
    for bx in (0, 1):
        for by in (0, 1):
            for bz in (0, 1):
                idxs.append(b[bx * 2 + by] + (iz1 if bz else iz0))
                ws.append(wxy[bx * 2 + by] * wz[bz])
    return idxs, ws


_MESH = plsc.VectorSubcoreMesh(core_axis_name="c", subcore_axis_name="s")


@functools.partial(
    pl.kernel,
    out_type=jax.ShapeDtypeStruct((N * NCOL,), jnp.float32),
    mesh=_MESH,
    scratch_types=[
        pltpu.VMEM((PW,), jnp.float32),
        pltpu.VMEM((PW,), jnp.float32),
        pltpu.VMEM((PW,), jnp.float32),
        pltpu.VMEM((8 * CF,), jnp.int32),
        pltpu.VMEM((8 * CF,), jnp.int32),
        pltpu.VMEM((8 * CHUNK,), jnp.float32),
        pltpu.VMEM((8 * CHUNK,), jnp.float32),
        pltpu.VMEM((8 * CF,), jnp.float32),
        pltpu.VMEM((8 * CF,), jnp.float32),
        pltpu.VMEM((CHUNK * NCOL,), jnp.float32),
        pltpu.SemaphoreType.DMA,
        pltpu.SemaphoreType.DMA,
    ],
    compiler_params=pltpu.CompilerParams(needs_layout_passes=False),
)
def _sc_encode(*args):
    xs, ys, zs = args[0], args[1], args[2]
    tabs = args[3:3 + NLVL]
    out = args[3 + NLVL]
    (xb, yb, zb, idxb0, idxb1, wb0, wb1, rows0, rows1, outb,
     sem0, sem1) = args[4 + NLVL:]
    idxbs, wbs, rowss = [idxb0, idxb1], [wb0, wb1], [rows0, rows1]
    sems = [sem0, sem1]

    wid = lax.axis_index("c") * 16 + lax.axis_index("s")
    base = wid * PW
    pltpu.sync_copy(xs.at[pl.ds(base, PW)], xb)
    pltpu.sync_copy(ys.at[pl.ds(base, PW)], yb)
    pltpu.sync_copy(zs.at[pl.ds(base, PW)], zb)

    inv = 1.0 / (WORLD_MAX - WORLD_MIN)

    def _norm(j, carry):
        for b in (xb, yb, zb):
            v = b[pl.ds(j * 16, 16)]
            b[pl.ds(j * 16, 16)] = (v - WORLD_MIN) * inv
        return carry

    lax.fori_loop(0, PW // 16, _norm, 0)

    def _coff(lvl):
        return [((2 * lvl + f) >> 3) * (8 * CHUNK)
                + ((2 * lvl + f) & 7) * 128 for f in range(NFEAT)]

    _DMA_LVLS = [lvl for lvl in range(NLVL) if not _STAGED[lvl]]
    ND = len(_DMA_LVLS)

    def _chunk(ch, carry):
        cb = ch * CHUNK

        def _phase_a(k):
            lvl = _DMA_LVLS[k]
            g, dense = _GRIDS[lvl], _IS_DENSE[lvl]
            idxb, wb = idxbs[k % 2], wbs[k % 2]

            def _grp_a(i, c, g=g, dense=dense, cb=cb, idxb=idxb, wb=wb):
                off = cb + i * 16
                x16 = xb[pl.ds(off, 16)]
                y16 = yb[pl.ds(off, 16)]
                z16 = zb[pl.ds(off, 16)]
                if dense:
                    idxs, ws = _dense_idx_w(x16, y16, z16, g)
                else:
                    idxs, ws = _hash_idx_w(x16, y16, z16, float(g))
                for m in range(8):
                    idxb[pl.ds(m * CF + i * 16, 16)] = idxs[m]
                    idxb[pl.ds(m * CF + CHUNK + i * 16, 16)] = idxs[m] + 128
                    wb[pl.ds(m * CHUNK + i * 16, 16)] = ws[m]
                return c

            lax.fori_loop(0, NGRP, _grp_a, 0)

        def _fire(k):
            lvl = _DMA_LVLS[k]
            idxb, rows = idxbs[k % 2], rowss[k % 2]
            return [
                pltpu.async_copy(
                    tabs[lvl].at[idxb.at[pl.ds(m * CF, CF)]],
                    rows.at[pl.ds(m * CF, CF)],
                    sems[k % 2])
                for m in range(8)
            ]

        def _acc(k):
            lvl = _DMA_LVLS[k]
            coff = _coff(lvl)
            wb, rows = wbs[k % 2], rowss[k % 2]

            def _grp_acc(i, c, coff=coff, wb=wb, rows=rows):
                ob = (i >> 3) * 1024 + (i & 7) * 16
                w16 = [wb[pl.ds(m * CHUNK + i * 16, 16)] for m in range(8)]
                for f in range(NFEAT):
                    acc = None
                    for m in range(8):
                        val = rows[pl.ds(m * CF + f * CHUNK + i * 16, 16)]
                        acc = (val * w16[m] if acc is None
                               else acc + val * w16[m])
                    outb[pl.ds(ob + coff[f], 16)] = acc
                return c

            lax.fori_loop(0, NGRP, _grp_acc, 0)

        _phase_a(0)
        pending = {0: _fire(0)}
        _phase_a(1)
        for k in range(ND):
            if k + 1 < ND:
                pending[k + 1] = _fire(k + 1)
            for cpy in pending.pop(k):
                cpy.wait()
            _acc(k)
            if k + 2 < ND:
                _phase_a(k + 2)

        p0 = base + cb
        for cblk in range(NCOL // 8):
            pltpu.sync_copy(
                outb.at[pl.ds(cblk * (8 * CHUNK), 8 * CHUNK)],
                out.at[pl.ds(cblk * (N * 8) + p0 * 8, 8 * CHUNK)])
        return carry

    lax.fori_loop(0, NCH, _chunk, 0)


def kernel(x, tables):
    xs = x[:, 0]
    ys = x[:, 1]
    zs = x[:, 2]
    flat = []
    for t, g, d in zip(tables, _GRIDS, _IS_DENSE):
        if d:
            flat.append(jnp.pad(t.transpose(0, 1, 3, 2),
                                ((0, 0), (0, 0), (0, 0), (0, 128 - g)))
                        .reshape(-1))
        else:
            flat.append(t.reshape(HASH_SIZE // 128, 128, NFEAT)
                         .transpose(0, 2, 1).reshape(-1))
    out = _sc_encode(xs, ys, zs, *flat)
    return (out.reshape(NCOL // 8, N // 128, 8, 128)
               .transpose(1, 3, 0, 2).reshape(N, NCOL))
